# baseline (speedup 1.0000x reference)
import functools

import jax
import jax.numpy as jnp
from jax import lax
from jax.experimental import pallas as pl
from jax.experimental.pallas import tpu as pltpu
from jax.experimental.pallas import tpu_sc as plsc

S = 8
G = 4
K = 1024
D = 96
T = 784
TP = 800
NC, NS = 2, 16
NW = NC * NS
ROWS_W = (G * TP) // NW


def _scores_argmin_body(g, x_ref, cb_ref, nn_ref, fidx_ref):
    for sl in range(G):
        xs = x_ref[:, D * sl:D * (sl + 1)]
        cs = cb_ref[sl]
        cn = jnp.sum(cs * cs, axis=1, keepdims=True)
        prod = lax.dot_general(
            cs, xs, (((1,), (1,)), ((), ())),
            preferred_element_type=jnp.float32,
            precision=lax.Precision.HIGHEST,
        )
        sc = cn - 2.0 * prod
        m = jnp.min(sc, axis=0, keepdims=True)
        kio = lax.broadcasted_iota(jnp.int32, sc.shape, 0)
        hit = jnp.where(sc == m, kio, jnp.int32(1 << 30))
        idx = jnp.min(hit, axis=0, keepdims=True)
        nn_ref[sl] = idx
        base = K * (G * g + sl)
        fidx_ref[sl, :, :T] = idx + jnp.int32(base)
        fidx_ref[sl, :, T:] = jnp.full((1, TP - T), base, jnp.int32)


def _assemble_body(x_ref, qa_ref, qb_ref, out_q_ref, out_l_ref):
    for s in range(S):
        q_ref = qa_ref if s < G else qb_ref
        qs = q_ref[s % G, :T, :]
        xs = x_ref[:, D * s:D * (s + 1)]
        r = qs - xs
        qq = xs + r
        out_q_ref[:, D * s:D * (s + 1)] = qq
        out_l_ref[:, D * s:D * (s + 1)] = (qq - xs) * (qq - xs)


def _sc_gather_body(fidx_hbm, table_hbm, out_hbm, idx_v, rows_v, sem):
    w = lax.axis_index("s") * NC + lax.axis_index("c")
    pltpu.sync_copy(fidx_hbm.at[w], idx_v)
    pltpu.async_copy(table_hbm.at[idx_v], rows_v, sem).wait()
    pltpu.sync_copy(rows_v, out_hbm.at[w])


@functools.cache
def _make_sc_gather():
    return pl.kernel(
        _sc_gather_body,
        out_type=jax.ShapeDtypeStruct((NW, ROWS_W, D), jnp.float32),
        mesh=plsc.VectorSubcoreMesh(core_axis_name="c", subcore_axis_name="s"),
        compiler_params=pltpu.CompilerParams(use_tc_tiling_on_sc=False),
        scratch_types=[
            pltpu.VMEM((ROWS_W,), jnp.int32),
            pltpu.VMEM((ROWS_W, D), jnp.float32),
            pltpu.SemaphoreType.DMA,
        ],
    )


def _scores_call(g, x2, codebooks):
    return pl.pallas_call(
        functools.partial(_scores_argmin_body, g),
        grid=(1,),
        in_specs=[
            pl.BlockSpec((T, G * D), lambda i, g=g: (0, g)),
            pl.BlockSpec((G, K, D), lambda i, g=g: (g, 0, 0)),
        ],
        out_specs=(
            pl.BlockSpec((G, 1, T), lambda i: (0, 0, 0)),
            pl.BlockSpec((G, 1, TP), lambda i: (0, 0, 0)),
        ),
        out_shape=(
            jax.ShapeDtypeStruct((G, 1, T), jnp.int32),
            jax.ShapeDtypeStruct((G, 1, TP), jnp.int32),
        ),
    )(x2, codebooks)


def kernel(inputs, codebooks):
    x2 = inputs.reshape(T, S * D)
    table = codebooks.reshape(S * K, D)
    gather = _make_sc_gather()

    nn_a, fidx_a = _scores_call(0, x2, codebooks)
    qa = gather(fidx_a.reshape(NW, ROWS_W), table)
    nn_b, fidx_b = _scores_call(1, x2, codebooks)
    qb = gather(fidx_b.reshape(NW, ROWS_W), table)

    nn_idx = jnp.concatenate([nn_a, nn_b], axis=0).reshape(S, 4, 196)
    q2, loss2 = pl.pallas_call(
        _assemble_body,
        out_shape=(
            jax.ShapeDtypeStruct((T, S * D), jnp.float32),
            jax.ShapeDtypeStruct((T, S * D), jnp.float32),
        ),
    )(x2, qa.reshape(G, TP, D), qb.reshape(G, TP, D))
    quantized = q2.reshape(1, 4, 196, S * D)
    loss = loss2.reshape(1, 4, 196, S * D)
    return (quantized, loss, nn_idx, codebooks)


# speedup vs baseline: 1.0743x; 1.0743x over previous
import functools

import jax
import jax.numpy as jnp
from jax import lax
from jax.experimental import pallas as pl
from jax.experimental.pallas import tpu as pltpu
from jax.experimental.pallas import tpu_sc as plsc

S = 8
NG = 2
G = S // NG
K = 1024
D = 96
DP = 128
B = 4
TT = 196
T = B * TT
NC, NS = 2, 16
NW = NC * NS
WPS = NW // G
TPp = 832
CW = TPp // WPS


def _scores_argmin_body(x_ref, cb_ref, nn_ref, fidx_ref, table_ref):
    for sl in range(G):
        xs = x_ref[:, D * sl:D * (sl + 1)]
        cs = cb_ref[sl]
        table_ref[K * sl:K * (sl + 1), :D] = cs
        cn = jnp.sum(cs * cs, axis=1, keepdims=True)
        prod = lax.dot_general(
            cs, xs, (((1,), (1,)), ((), ())),
            preferred_element_type=jnp.float32,
            precision=lax.Precision.HIGHEST,
        )
        sc = cn - 2.0 * prod
        m = jnp.min(sc, axis=0, keepdims=True)
        kio = lax.broadcasted_iota(jnp.int32, sc.shape, 0)
        hit = jnp.where(sc == m, kio, jnp.int32(1 << 30))
        idx = jnp.min(hit, axis=0, keepdims=True)
        nn_ref[sl] = idx
        full = jnp.concatenate(
            [idx + jnp.int32(K * sl),
             jnp.full((1, TPp - T), K * sl, jnp.int32)], axis=1)
        for j in range(WPS):
            fidx_ref[WPS * sl + j] = full[:, CW * j:CW * (j + 1)]


def _sc_gather_body(fidx_hbm, table_hbm, out_hbm, idx_v, rows_v, sem):
    w = lax.axis_index("s") * NC + lax.axis_index("c")
    sl = w // WPS
    wi = w % WPS
    pltpu.sync_copy(fidx_hbm.at[w, 0], idx_v)
    pltpu.async_copy(table_hbm.at[idx_v], rows_v, sem).wait()
    pltpu.sync_copy(rows_v, out_hbm.at[sl, pl.ds(CW * wi, CW)])


def _assemble_body(x_ref, qa_ref, qb_ref, nna_ref, nnb_ref,
                   q4_ref, l4_ref, nn_ref):
    qg = (qa_ref, qb_ref)
    ng = (nna_ref, nnb_ref)
    for s in range(S):
        grp, sl = divmod(s, G)
        qs = qg[grp][sl, :T, :D]
        xs = x_ref[:, D * s:D * (s + 1)]
        r = qs - xs
        qq = xs + r
        ls = (qq - xs) * (qq - xs)
        for b in range(B):
            q4_ref[0, b, :, D * s:D * (s + 1)] = qq[TT * b:TT * (b + 1), :]
            l4_ref[0, b, :, D * s:D * (s + 1)] = ls[TT * b:TT * (b + 1), :]
        row = ng[grp][sl]
        nn_ref[s] = jnp.concatenate(
            [row[:, TT * b:TT * (b + 1)] for b in range(B)], axis=0)


@functools.cache
def _make_sc_gather():
    return pl.kernel(
        _sc_gather_body,
        out_type=jax.ShapeDtypeStruct((G, TPp, DP), jnp.float32),
        mesh=plsc.VectorSubcoreMesh(core_axis_name="c", subcore_axis_name="s"),
        scratch_types=[
            pltpu.VMEM((CW,), jnp.int32),
            pltpu.VMEM((CW, DP), jnp.float32),
            pltpu.SemaphoreType.DMA,
        ],
    )


def _scores_call(g, x2, codebooks):
    return pl.pallas_call(
        _scores_argmin_body,
        grid=(1,),
        in_specs=[
            pl.BlockSpec((T, G * D), lambda i, g=g: (0, g)),
            pl.BlockSpec((G, K, D), lambda i, g=g: (g, 0, 0)),
        ],
        out_specs=(
            pl.BlockSpec((G, 1, T), lambda i: (0, 0, 0)),
            pl.BlockSpec((NW, 1, CW), lambda i: (0, 0, 0)),
            pl.BlockSpec((G * K, DP), lambda i: (0, 0)),
        ),
        out_shape=(
            jax.ShapeDtypeStruct((G, 1, T), jnp.int32),
            jax.ShapeDtypeStruct((NW, 1, CW), jnp.int32),
            jax.ShapeDtypeStruct((G * K, DP), jnp.float32),
        ),
    )(x2, codebooks)


def kernel(inputs, codebooks):
    x2 = inputs.reshape(T, S * D)
    gather = _make_sc_gather()

    nn_a, fidx_a, table_a = _scores_call(0, x2, codebooks)
    qa = gather(fidx_a, table_a)
    nn_b, fidx_b, table_b = _scores_call(1, x2, codebooks)
    qb = gather(fidx_b, table_b)

    q4, l4, nn_idx = pl.pallas_call(
        _assemble_body,
        out_shape=(
            jax.ShapeDtypeStruct((1, B, TT, S * D), jnp.float32),
            jax.ShapeDtypeStruct((1, B, TT, S * D), jnp.float32),
            jax.ShapeDtypeStruct((S, B, TT), jnp.int32),
        ),
    )(x2, qa, qb, nn_a, nn_b)
    return (q4, l4, nn_idx, codebooks)


# speedup vs baseline: 1.0783x; 1.0037x over previous
import functools

import jax
import jax.numpy as jnp
from jax import lax
from jax.experimental import pallas as pl
from jax.experimental.pallas import tpu as pltpu
from jax.experimental.pallas import tpu_sc as plsc

S = 8
NG = 2
G = S // NG
K = 1024
D = 96
DP = 128
B = 4
TT = 196
T = B * TT
NC, NS = 2, 16
NW = NC * NS
WPS = NW // G
TPp = 832
CW = TPp // WPS


def _scores_argmin_body(x_ref, cb_ref, nn_ref, fidx_ref, table_ref):
    xall = jnp.concatenate([x_ref[b] for b in range(B)], axis=0)
    for sl in range(G):
        xs = xall[:, D * sl:D * (sl + 1)]
        cs = cb_ref[sl]
        table_ref[K * sl:K * (sl + 1), :D] = cs
        cn = jnp.sum(cs * cs, axis=1, keepdims=True)
        prod = lax.dot_general(
            cs, xs, (((1,), (1,)), ((), ())),
            preferred_element_type=jnp.float32,
            precision=lax.Precision.HIGHEST,
        )
        sc = cn - 2.0 * prod
        m = jnp.min(sc, axis=0, keepdims=True)
        kio = lax.broadcasted_iota(jnp.int32, sc.shape, 0)
        hit = jnp.where(sc == m, kio, jnp.int32(1 << 30))
        idx = jnp.min(hit, axis=0, keepdims=True)
        nn_ref[sl] = idx
        full = jnp.concatenate(
            [idx + jnp.int32(K * sl),
             jnp.full((1, TPp - T), K * sl, jnp.int32)], axis=1)
        for j in range(WPS):
            fidx_ref[WPS * sl + j] = full[:, CW * j:CW * (j + 1)]


def _sc_gather_body(fidx_hbm, table_hbm, out_hbm, idx_v, rows_v, sem):
    w = lax.axis_index("s") * NC + lax.axis_index("c")
    sl = w // WPS
    wi = w % WPS
    pltpu.sync_copy(fidx_hbm.at[w, 0], idx_v)
    pltpu.async_copy(table_hbm.at[idx_v], rows_v, sem).wait()
    pltpu.sync_copy(rows_v, out_hbm.at[sl, pl.ds(CW * wi, CW)])


def _assemble_body(x_ref, qa_ref, qb_ref, nna_ref, nnb_ref,
                   q4_ref, l4_ref, nn_ref):
    qg = (qa_ref, qb_ref)
    ng = (nna_ref, nnb_ref)
    for s in range(S):
        grp, sl = divmod(s, G)
        for b in range(B):
            qs = qg[grp][sl, TT * b:TT * (b + 1), :D]
            xs = x_ref[b, :, D * s:D * (s + 1)]
            r = qs - xs
            qq = xs + r
            q4_ref[0, b, :, D * s:D * (s + 1)] = qq
            l4_ref[0, b, :, D * s:D * (s + 1)] = (qq - xs) * (qq - xs)
        row = ng[grp][sl]
        nn_ref[s] = jnp.concatenate(
            [row[:, TT * b:TT * (b + 1)] for b in range(B)], axis=0)


@functools.cache
def _make_sc_gather():
    return pl.kernel(
        _sc_gather_body,
        out_type=jax.ShapeDtypeStruct((G, TPp, DP), jnp.float32),
        mesh=plsc.VectorSubcoreMesh(core_axis_name="c", subcore_axis_name="s"),
        scratch_types=[
            pltpu.VMEM((CW,), jnp.int32),
            pltpu.VMEM((CW, DP), jnp.float32),
            pltpu.SemaphoreType.DMA,
        ],
    )


def _scores_call(g, x4, codebooks):
    return pl.pallas_call(
        _scores_argmin_body,
        grid=(1,),
        in_specs=[
            pl.BlockSpec((B, TT, G * D), lambda i, g=g: (0, 0, g)),
            pl.BlockSpec((G, K, D), lambda i, g=g: (g, 0, 0)),
        ],
        out_specs=(
            pl.BlockSpec((G, 1, T), lambda i: (0, 0, 0)),
            pl.BlockSpec((NW, 1, CW), lambda i: (0, 0, 0)),
            pl.BlockSpec((G * K, DP), lambda i: (0, 0)),
        ),
        out_shape=(
            jax.ShapeDtypeStruct((G, 1, T), jnp.int32),
            jax.ShapeDtypeStruct((NW, 1, CW), jnp.int32),
            jax.ShapeDtypeStruct((G * K, DP), jnp.float32),
        ),
    )(x4, codebooks)


def kernel(inputs, codebooks):
    gather = _make_sc_gather()

    nn_a, fidx_a, table_a = _scores_call(0, inputs, codebooks)
    qa = gather(fidx_a, table_a)
    nn_b, fidx_b, table_b = _scores_call(1, inputs, codebooks)
    qb = gather(fidx_b, table_b)

    q4, l4, nn_idx = pl.pallas_call(
        _assemble_body,
        out_shape=(
            jax.ShapeDtypeStruct((1, B, TT, S * D), jnp.float32),
            jax.ShapeDtypeStruct((1, B, TT, S * D), jnp.float32),
            jax.ShapeDtypeStruct((S, B, TT), jnp.int32),
        ),
    )(inputs, qa, qb, nn_a, nn_b)
    return (q4, l4, nn_idx, codebooks)
